# Initial kernel scaffold; baseline (speedup 1.0000x reference)
import jax
import jax.numpy as jnp
from jax.experimental import pallas as pl


def kernel(evaluation, template, ew1, eb1, ew2, eb2, ew3, eb3, elin_w, elin_b, egru_wih, egru_whh, egru_bih, egru_bhh, tw1, tb1, tw2, tb2, tw3, tb3, tlin_w, tlin_b, tgru_wih, tgru_whh, tgru_bih, tgru_bhh, attn_w, attn_b, hid_w, hid_b, cls_w, cls_b):
    raise NotImplementedError("write your pallas kernel here")


# speedup vs baseline: 2.7222x; 2.7222x over previous
import jax
import jax.numpy as jnp
from jax.experimental import pallas as pl
from jax.experimental.pallas import tpu as pltpu

F32 = jnp.float32


def _compose_conv_weights(w1, b1, w2, b2, w3, b3):
    k12 = jnp.zeros((w2.shape[0], w1.shape[1], 9, 9), F32)
    for dy in range(5):
        for dx in range(5):
            k12 = k12.at[:, :, dy:dy + 5, dx:dx + 5].add(
                jnp.einsum('om,miuv->oiuv', w2[:, :, dy, dx], w1))
    k13 = jnp.zeros((w3.shape[0], w1.shape[1], 13, 13), F32)
    for dy in range(5):
        for dx in range(5):
            k13 = k13.at[:, :, dy:dy + 9, dx:dx + 9].add(
                jnp.einsum('om,miuv->oiuv', w3[:, :, dy, dx], k12))
    c2 = b2 + w2.sum((2, 3)) @ b1
    c3 = b3 + w3.sum((2, 3)) @ c2
    return k13[:, 0].reshape(w3.shape[0], 169), c3


def _conv_pool_kernel(img_ref, w_ref, b_ref, out_ref):
    w = w_ref[0]
    b = b_ref[0]
    base = pl.multiple_of(16 * pl.program_id(1), 16)
    rows = img_ref[0, pl.dslice(base, 32), :]
    conv_rows = []
    for j in range(16):
        taps = []
        for dy in range(13):
            row = rows[j + dy]
            for dx in range(13):
                taps.append(row[dx:dx + 1028])
        patches = jnp.stack(taps, axis=0)
        y = jax.lax.dot_general(w, patches, (((1,), (0,)), ((), ())),
                                preferred_element_type=F32)
        conv_rows.append(y + b.T)
    pooled = []
    for q in range(8):
        p = jnp.maximum(conv_rows[2 * q], conv_rows[2 * q + 1])
        m = p[:, 0:1024]
        for k in range(1, 5):
            m = jnp.maximum(m, p[:, k:k + 1024])
        pooled.append(m)
    out_ref[0] = jnp.stack(pooled, axis=1)


def _linear_kernel(x_ref, w_ref, b_ref, out_ref):
    x = x_ref[0]
    w = w_ref[0]
    out_ref[0] = jnp.dot(x, w, preferred_element_type=F32) + b_ref[0]


def _gru_kernel(x_ref, wih_ref, whh_ref, bih_ref, bhh_ref, out_ref):
    wih = wih_ref[0]
    whh = whh_ref[0]
    bih = bih_ref[0]
    bhh = bhh_ref[0]
    h = jnp.zeros((1024, 64), F32)
    for i in range(8):
        xt = x_ref[0, i]
        gi = jnp.dot(xt, wih, preferred_element_type=F32) + bih
        gh = jnp.dot(h, whh, preferred_element_type=F32) + bhh
        r = jax.nn.sigmoid(gi[:, 0:64] + gh[:, 0:64])
        z = jax.nn.sigmoid(gi[:, 64:128] + gh[:, 64:128])
        n = jnp.tanh(gi[:, 128:192] + r * gh[:, 128:192])
        h = (1.0 - z) * n + z * h
        out_ref[0, i] = h


def _attn_kernel(tp_ref, ev_ref, aw_ref, ab_ref, hw_ref, hb_ref,
                 cw_ref, cb_ref, out_ref):
    tp = tp_ref[0]
    ev = ev_ref[0]
    s = jax.lax.dot_general(tp, ev, (((1,), (1,)), ((), ())),
                            preferred_element_type=F32)
    s = s - jnp.max(s, axis=1, keepdims=True)
    e = jnp.exp(s)
    a = e / jnp.sum(e, axis=1, keepdims=True)
    x = jax.lax.dot_general(a, tp, (((0,), (0,)), ((), ())),
                            preferred_element_type=F32)
    y = jnp.abs(x - ev)
    al = jnp.dot(ev, aw_ref[...], preferred_element_type=F32) + ab_ref[0, 0]
    al = al - jnp.max(al, axis=0, keepdims=True)
    ae = jnp.exp(al)
    attn = ae / jnp.sum(ae, axis=0, keepdims=True)
    pooled = jax.lax.dot_general(attn, y, (((0,), (0,)), ((), ())),
                                 preferred_element_type=F32)
    h = jax.nn.relu(jnp.dot(pooled, hw_ref[...],
                            preferred_element_type=F32) + hb_ref[0])
    lg = jnp.dot(h, cw_ref[...], preferred_element_type=F32) + cb_ref[0]
    lg = lg - jnp.max(lg, axis=1, keepdims=True)
    le = jnp.exp(lg)
    out_ref[0] = le / jnp.sum(le, axis=1, keepdims=True)


def kernel(evaluation, template, ew1, eb1, ew2, eb2, ew3, eb3, elin_w, elin_b,
           egru_wih, egru_whh, egru_bih, egru_bhh,
           tw1, tb1, tw2, tb2, tw3, tb3, tlin_w, tlin_b,
           tgru_wih, tgru_whh, tgru_bih, tgru_bhh,
           attn_w, attn_b, hid_w, hid_b, cls_w, cls_b):
    B = evaluation.shape[0]

    we, be = _compose_conv_weights(ew1, eb1, ew2, eb2, ew3, eb3)
    wt, bt = _compose_conv_weights(tw1, tb1, tw2, tb2, tw3, tb3)
    wc = jnp.stack([we, wt])
    bc = jnp.stack([be, bt])[:, None, :]
    imgs = jnp.concatenate([evaluation, template], axis=0)
    imgs = jnp.pad(imgs, ((0, 0), (0, 16), (0, 0)))

    pooled = pl.pallas_call(
        _conv_pool_kernel,
        grid=(2 * B, 8),
        in_specs=[
            pl.BlockSpec((1, 144, 1040), lambda b, r: (b, 0, 0)),
            pl.BlockSpec((1, 64, 169), lambda b, r: (b // B, 0, 0)),
            pl.BlockSpec((1, 1, 64), lambda b, r: (b // B, 0, 0)),
        ],
        out_specs=pl.BlockSpec((1, 64, 8, 1024), lambda b, r: (b, 0, r, 0)),
        out_shape=jax.ShapeDtypeStruct((2 * B, 64, 64, 1024), F32),
        compiler_params=pltpu.CompilerParams(
            dimension_semantics=("parallel", "arbitrary")),
    )(imgs, wc, bc)

    xlin = pooled[:, :, :58, :].reshape(2 * B, 1024, 3712)
    lw = jnp.stack([elin_w.T, tlin_w.T])
    lb = jnp.stack([elin_b, tlin_b])[:, None, :]

    feats = pl.pallas_call(
        _linear_kernel,
        grid=(2 * B,),
        in_specs=[
            pl.BlockSpec((1, 1024, 3712), lambda b: (b, 0, 0)),
            pl.BlockSpec((1, 3712, 64), lambda b: (b // B, 0, 0)),
            pl.BlockSpec((1, 1, 64), lambda b: (b // B, 0, 0)),
        ],
        out_specs=pl.BlockSpec((1, 1024, 64), lambda b: (b, 0, 0)),
        out_shape=jax.ShapeDtypeStruct((2 * B, 1024, 64), F32),
        compiler_params=pltpu.CompilerParams(
            dimension_semantics=("parallel",)),
    )(xlin, lw, lb)

    gx = feats.reshape(2, B, 1024, 64)
    wih = jnp.stack([egru_wih.T, tgru_wih.T])
    whh = jnp.stack([egru_whh.T, tgru_whh.T])
    bih = jnp.stack([egru_bih, tgru_bih])[:, None, :]
    bhh = jnp.stack([egru_bhh, tgru_bhh])[:, None, :]

    ys = pl.pallas_call(
        _gru_kernel,
        grid=(2,),
        in_specs=[
            pl.BlockSpec((1, B, 1024, 64), lambda p: (p, 0, 0, 0)),
            pl.BlockSpec((1, 64, 192), lambda p: (p, 0, 0)),
            pl.BlockSpec((1, 64, 192), lambda p: (p, 0, 0)),
            pl.BlockSpec((1, 1, 192), lambda p: (p, 0, 0)),
            pl.BlockSpec((1, 1, 192), lambda p: (p, 0, 0)),
        ],
        out_specs=pl.BlockSpec((1, B, 1024, 64), lambda p: (p, 0, 0, 0)),
        out_shape=jax.ShapeDtypeStruct((2, B, 1024, 64), F32),
        compiler_params=pltpu.CompilerParams(
            dimension_semantics=("parallel",)),
    )(gx, wih, whh, bih, bhh)

    ev = ys[0]
    tp = ys[1]

    out = pl.pallas_call(
        _attn_kernel,
        grid=(B,),
        in_specs=[
            pl.BlockSpec((1, 1024, 64), lambda b: (b, 0, 0)),
            pl.BlockSpec((1, 1024, 64), lambda b: (b, 0, 0)),
            pl.BlockSpec((64, 1), lambda b: (0, 0)),
            pl.BlockSpec((1, 1), lambda b: (0, 0)),
            pl.BlockSpec((64, 128), lambda b: (0, 0)),
            pl.BlockSpec((1, 128), lambda b: (0, 0)),
            pl.BlockSpec((128, 2), lambda b: (0, 0)),
            pl.BlockSpec((1, 2), lambda b: (0, 0)),
        ],
        out_specs=pl.BlockSpec((1, 1, 2), lambda b: (b, 0, 0)),
        out_shape=jax.ShapeDtypeStruct((B, 1, 2), F32),
        compiler_params=pltpu.CompilerParams(
            dimension_semantics=("parallel",)),
    )(tp, ev, attn_w.T, attn_b[None, :], hid_w.T, hid_b[None, :],
      cls_w.T, cls_b[None, :])

    return out.reshape(B, 2)


# speedup vs baseline: 3.3540x; 1.2321x over previous
import jax
import jax.numpy as jnp
from jax.experimental import pallas as pl
from jax.experimental.pallas import tpu as pltpu

F32 = jnp.float32


def _compose_conv_weights(w1, b1, w2, b2, w3, b3):
    k12 = jnp.zeros((w2.shape[0], w1.shape[1], 9, 9), F32)
    for dy in range(5):
        for dx in range(5):
            k12 = k12.at[:, :, dy:dy + 5, dx:dx + 5].add(
                jnp.einsum('om,miuv->oiuv', w2[:, :, dy, dx], w1))
    k13 = jnp.zeros((w3.shape[0], w1.shape[1], 13, 13), F32)
    for dy in range(5):
        for dx in range(5):
            k13 = k13.at[:, :, dy:dy + 9, dx:dx + 9].add(
                jnp.einsum('om,miuv->oiuv', w3[:, :, dy, dx], k12))
    c2 = b2 + w2.sum((2, 3)) @ b1
    c3 = b3 + w3.sum((2, 3)) @ c2
    return k13[:, 0].reshape(w3.shape[0], 169), c3


def _conv_pool_kernel(img_ref, w_ref, b_ref, out_ref):
    w = w_ref[0]
    b = b_ref[0]
    base = pl.multiple_of(16 * pl.program_id(1), 16)
    rows = img_ref[0, pl.dslice(base, 32), :]
    zero = jnp.zeros((1028,), F32)
    shifted = []
    for d in range(28):
        row = rows[d]
        shifted.append(jnp.stack(
            [row[dx:dx + 1028] for dx in range(13)] + [zero] * 3, axis=0))
    t_all = jnp.stack(shifted, axis=0)
    conv_rows = []
    for j in range(16):
        patches = t_all[j:j + 13].reshape(208, 1028)
        y = jax.lax.dot_general(w, patches, (((1,), (0,)), ((), ())),
                                preferred_element_type=F32)
        conv_rows.append(y + b.T)
    pooled = []
    for q in range(8):
        p = jnp.maximum(conv_rows[2 * q], conv_rows[2 * q + 1])
        m = p[:, 0:1024]
        for k in range(1, 5):
            m = jnp.maximum(m, p[:, k:k + 1024])
        pooled.append(m)
    out_ref[0] = jnp.stack(pooled, axis=1)


def _linear_kernel(x_ref, w_ref, b_ref, out_ref):
    x = x_ref[0]
    w = w_ref[0]
    out_ref[0] = jnp.dot(x, w, preferred_element_type=F32) + b_ref[0]


def _gru_kernel(x_ref, wih_ref, whh_ref, bih_ref, bhh_ref, out_ref):
    wih = wih_ref[0]
    whh = whh_ref[0]
    bih = bih_ref[0]
    bhh = bhh_ref[0]
    h = jnp.zeros((1024, 64), F32)
    for i in range(8):
        xt = x_ref[0, i]
        gi = jnp.dot(xt, wih, preferred_element_type=F32) + bih
        gh = jnp.dot(h, whh, preferred_element_type=F32) + bhh
        r = jax.nn.sigmoid(gi[:, 0:64] + gh[:, 0:64])
        z = jax.nn.sigmoid(gi[:, 64:128] + gh[:, 64:128])
        n = jnp.tanh(gi[:, 128:192] + r * gh[:, 128:192])
        h = (1.0 - z) * n + z * h
        out_ref[0, i] = h


def _attn_kernel(tp_ref, ev_ref, aw_ref, ab_ref, hw_ref, hb_ref,
                 cw_ref, cb_ref, out_ref):
    tp = tp_ref[0]
    ev = ev_ref[0]
    s = jax.lax.dot_general(tp, ev, (((1,), (1,)), ((), ())),
                            preferred_element_type=F32)
    s = s - jnp.max(s, axis=1, keepdims=True)
    e = jnp.exp(s)
    a = e / jnp.sum(e, axis=1, keepdims=True)
    x = jax.lax.dot_general(a, tp, (((0,), (0,)), ((), ())),
                            preferred_element_type=F32)
    y = jnp.abs(x - ev)
    al = jnp.dot(ev, aw_ref[...], preferred_element_type=F32) + ab_ref[0, 0]
    al = al - jnp.max(al, axis=0, keepdims=True)
    ae = jnp.exp(al)
    attn = ae / jnp.sum(ae, axis=0, keepdims=True)
    pooled = jax.lax.dot_general(attn, y, (((0,), (0,)), ((), ())),
                                 preferred_element_type=F32)
    h = jax.nn.relu(jnp.dot(pooled, hw_ref[...],
                            preferred_element_type=F32) + hb_ref[0])
    lg = jnp.dot(h, cw_ref[...], preferred_element_type=F32) + cb_ref[0]
    lg = lg - jnp.max(lg, axis=1, keepdims=True)
    le = jnp.exp(lg)
    out_ref[0] = le / jnp.sum(le, axis=1, keepdims=True)


def kernel(evaluation, template, ew1, eb1, ew2, eb2, ew3, eb3, elin_w, elin_b,
           egru_wih, egru_whh, egru_bih, egru_bhh,
           tw1, tb1, tw2, tb2, tw3, tb3, tlin_w, tlin_b,
           tgru_wih, tgru_whh, tgru_bih, tgru_bhh,
           attn_w, attn_b, hid_w, hid_b, cls_w, cls_b):
    B = evaluation.shape[0]

    we, be = _compose_conv_weights(ew1, eb1, ew2, eb2, ew3, eb3)
    wt, bt = _compose_conv_weights(tw1, tb1, tw2, tb2, tw3, tb3)
    wc = jnp.stack([we, wt])
    wc = jnp.pad(wc.reshape(2, 64, 13, 13),
                 ((0, 0), (0, 0), (0, 0), (0, 3))).reshape(2, 64, 208)
    bc = jnp.stack([be, bt])[:, None, :]
    imgs = jnp.concatenate([evaluation, template], axis=0)
    imgs = jnp.pad(imgs, ((0, 0), (0, 16), (0, 0)))

    pooled = pl.pallas_call(
        _conv_pool_kernel,
        grid=(2 * B, 8),
        in_specs=[
            pl.BlockSpec((1, 144, 1040), lambda b, r: (b, 0, 0)),
            pl.BlockSpec((1, 64, 208), lambda b, r: (b // B, 0, 0)),
            pl.BlockSpec((1, 1, 64), lambda b, r: (b // B, 0, 0)),
        ],
        out_specs=pl.BlockSpec((1, 64, 8, 1024), lambda b, r: (b, 0, r, 0)),
        out_shape=jax.ShapeDtypeStruct((2 * B, 64, 64, 1024), F32),
        compiler_params=pltpu.CompilerParams(
            dimension_semantics=("parallel", "arbitrary")),
    )(imgs, wc, bc)

    xlin = pooled[:, :, :58, :].reshape(2 * B, 1024, 3712)
    lw = jnp.stack([elin_w.T, tlin_w.T])
    lb = jnp.stack([elin_b, tlin_b])[:, None, :]

    feats = pl.pallas_call(
        _linear_kernel,
        grid=(2 * B,),
        in_specs=[
            pl.BlockSpec((1, 1024, 3712), lambda b: (b, 0, 0)),
            pl.BlockSpec((1, 3712, 64), lambda b: (b // B, 0, 0)),
            pl.BlockSpec((1, 1, 64), lambda b: (b // B, 0, 0)),
        ],
        out_specs=pl.BlockSpec((1, 1024, 64), lambda b: (b, 0, 0)),
        out_shape=jax.ShapeDtypeStruct((2 * B, 1024, 64), F32),
        compiler_params=pltpu.CompilerParams(
            dimension_semantics=("parallel",)),
    )(xlin, lw, lb)

    gx = feats.reshape(2, B, 1024, 64)
    wih = jnp.stack([egru_wih.T, tgru_wih.T])
    whh = jnp.stack([egru_whh.T, tgru_whh.T])
    bih = jnp.stack([egru_bih, tgru_bih])[:, None, :]
    bhh = jnp.stack([egru_bhh, tgru_bhh])[:, None, :]

    ys = pl.pallas_call(
        _gru_kernel,
        grid=(2,),
        in_specs=[
            pl.BlockSpec((1, B, 1024, 64), lambda p: (p, 0, 0, 0)),
            pl.BlockSpec((1, 64, 192), lambda p: (p, 0, 0)),
            pl.BlockSpec((1, 64, 192), lambda p: (p, 0, 0)),
            pl.BlockSpec((1, 1, 192), lambda p: (p, 0, 0)),
            pl.BlockSpec((1, 1, 192), lambda p: (p, 0, 0)),
        ],
        out_specs=pl.BlockSpec((1, B, 1024, 64), lambda p: (p, 0, 0, 0)),
        out_shape=jax.ShapeDtypeStruct((2, B, 1024, 64), F32),
        compiler_params=pltpu.CompilerParams(
            dimension_semantics=("parallel",)),
    )(gx, wih, whh, bih, bhh)

    ev = ys[0]
    tp = ys[1]

    out = pl.pallas_call(
        _attn_kernel,
        grid=(B,),
        in_specs=[
            pl.BlockSpec((1, 1024, 64), lambda b: (b, 0, 0)),
            pl.BlockSpec((1, 1024, 64), lambda b: (b, 0, 0)),
            pl.BlockSpec((64, 1), lambda b: (0, 0)),
            pl.BlockSpec((1, 1), lambda b: (0, 0)),
            pl.BlockSpec((64, 128), lambda b: (0, 0)),
            pl.BlockSpec((1, 128), lambda b: (0, 0)),
            pl.BlockSpec((128, 2), lambda b: (0, 0)),
            pl.BlockSpec((1, 2), lambda b: (0, 0)),
        ],
        out_specs=pl.BlockSpec((1, 1, 2), lambda b: (b, 0, 0)),
        out_shape=jax.ShapeDtypeStruct((B, 1, 2), F32),
        compiler_params=pltpu.CompilerParams(
            dimension_semantics=("parallel",)),
    )(tp, ev, attn_w.T, attn_b[None, :], hid_w.T, hid_b[None, :],
      cls_w.T, cls_b[None, :])

    return out.reshape(B, 2)
